# Initial kernel scaffold; baseline (speedup 1.0000x reference)
import jax
import jax.numpy as jnp
from jax.experimental import pallas as pl


def kernel(x, edge_index, Wxr, bxr, Wxz, bxz, Wxn, bxn, Whr, bhr, Whz, bhz, Whn, bhn, Wfc, bfc):
    raise NotImplementedError("write your pallas kernel here")


# speedup vs baseline: 15.0138x; 15.0138x over previous
import functools

import jax
import jax.numpy as jnp
from jax import lax
from jax.experimental import pallas as pl
from jax.experimental.pallas import tpu as pltpu
from jax.experimental.pallas import tpu_sc as plsc

B, T, N, E = 2, 4, 10000, 320000
D = 128
NS = 16
NC = 2
EK = 80
NPAD = 10240
ZCH = NPAD // NS

_mesh = plsc.VectorSubcoreMesh(core_axis_name="c", subcore_axis_name="s")


def _deg_body(col_hbm, deg_hbm, colv, onesv, zbuf, deg_sh):
    c = lax.axis_index("c")
    s = lax.axis_index("s")
    for j in range(ZCH // 16):
        zbuf[pl.ds(j * 16, 16)] = jnp.zeros((16,), jnp.float32)
    for j in range(EK // 16):
        onesv[pl.ds(j * 16, 16)] = jnp.ones((16,), jnp.float32)
    pltpu.sync_copy(zbuf, deg_sh.at[pl.ds(s * ZCH, ZCH)])
    plsc.subcore_barrier()
    w = s * NC + c
    ept = E // (NS * NC)
    base = w * ept

    def step(g, carry):
        pltpu.sync_copy(col_hbm.at[pl.ds(base + g * EK, EK)], colv)
        pltpu.sync_copy(onesv, deg_sh.at[colv], add=True)
        return carry

    lax.fori_loop(0, ept // EK, step, 0)
    plsc.subcore_barrier()
    pltpu.sync_copy(deg_sh.at[pl.ds(s * ZCH, ZCH)],
                    deg_hbm.at[pl.ds(c * NPAD + s * ZCH, ZCH)])


_deg = pl.kernel(
    _deg_body,
    out_type=jax.ShapeDtypeStruct((NC * NPAD,), jnp.float32),
    mesh=_mesh,
    scratch_types=[
        pltpu.VMEM((EK,), jnp.int32),
        pltpu.VMEM((EK,), jnp.float32),
        pltpu.VMEM((ZCH,), jnp.float32),
        pltpu.VMEM_SHARED((NPAD,), jnp.float32),
    ],
)


def _prop_body(xs_hbm, rowb_hbm, col_hbm, out_hbm, rowv, colv, gbuf, out_sh):
    c = lax.axis_index("c")
    s = lax.axis_index("s")
    rpt = N // 5

    @pl.when(s < 5)
    def _():
        pltpu.sync_copy(xs_hbm.at[pl.ds(c * N + s * rpt, rpt)],
                        out_sh.at[pl.ds(s * rpt, rpt)])

    plsc.subcore_barrier()
    ept = E // NS
    base = s * ept

    def step(g, carry):
        off = base + g * EK
        pltpu.sync_copy(rowb_hbm.at[pl.ds(c * E + off, EK)], rowv)
        pltpu.sync_copy(col_hbm.at[pl.ds(off, EK)], colv)
        pltpu.sync_copy(xs_hbm.at[rowv], gbuf)
        pltpu.sync_copy(gbuf, out_sh.at[colv], add=True)
        return carry

    lax.fori_loop(0, ept // EK, step, 0)
    plsc.subcore_barrier()

    @pl.when(s < 5)
    def _():
        pltpu.sync_copy(out_sh.at[pl.ds(s * rpt, rpt)],
                        out_hbm.at[pl.ds(c * N + s * rpt, rpt)])


_prop = pl.kernel(
    _prop_body,
    out_type=jax.ShapeDtypeStruct((B * N, D), jnp.float32),
    mesh=_mesh,
    scratch_types=[
        pltpu.VMEM((EK,), jnp.int32),
        pltpu.VMEM((EK,), jnp.int32),
        pltpu.VMEM((EK, D), jnp.float32),
        pltpu.VMEM_SHARED((N, D), jnp.float32),
    ],
)


def _dinv_body(deg_ref, out_ref):
    out_ref[...] = lax.rsqrt(deg_ref[0] + deg_ref[1] + 1.0)


def _dinv(degs):
    return pl.pallas_call(
        _dinv_body,
        out_shape=jax.ShapeDtypeStruct((NPAD // D, D), jnp.float32),
    )(degs)


def _scale_body(x_ref, d_ref, o_ref):
    o_ref[0, 0] = x_ref[0, 0] * d_ref[...]


def _scale(xT, dinvb, rb):
    return pl.pallas_call(
        _scale_body,
        grid=(T, B, N // rb),
        in_specs=[
            pl.BlockSpec((1, 1, rb, D), lambda t, b, i: (t, b, i, 0)),
            pl.BlockSpec((rb, D), lambda t, b, i: (i, 0)),
        ],
        out_specs=pl.BlockSpec((1, 1, rb, D), lambda t, b, i: (t, b, i, 0)),
        out_shape=jax.ShapeDtypeStruct((T, B, N, D), jnp.float32),
    )(xT, dinvb)


def _gate_body(px, ph, h, dv, wx, bx, wh, bh, wf, bf, hn_o, hs_o, out_o):
    d = dv[...]
    u = jnp.dot(px[0] * d, wx[...], preferred_element_type=jnp.float32) + bx[...]
    v = jnp.dot(ph[0] * d, wh[...], preferred_element_type=jnp.float32) + bh[...]
    r = jax.nn.sigmoid(u[:, :D] + v[:, :D])
    z = jax.nn.sigmoid(u[:, D:2 * D] + v[:, D:2 * D])
    n = jnp.tanh(u[:, 2 * D:] + r * v[:, 2 * D:])
    hn = (1.0 - z) * h[0] + z * n
    hn_o[0] = hn
    hs_o[0] = hn * d
    out_o[0] = jnp.dot(hn, wf[...], preferred_element_type=jnp.float32) + bf[...]


def _gate(Px, Ph, h, dinvb, Wx, bx, Wh, bh, Wf, bf, rb):
    node = pl.BlockSpec((1, rb, D), lambda b, i: (b, i, 0))
    return pl.pallas_call(
        _gate_body,
        grid=(B, N // rb),
        in_specs=[
            node, node, node,
            pl.BlockSpec((rb, D), lambda b, i: (i, 0)),
            pl.BlockSpec((D, 3 * D), lambda b, i: (0, 0)),
            pl.BlockSpec((1, 3 * D), lambda b, i: (0, 0)),
            pl.BlockSpec((D, 3 * D), lambda b, i: (0, 0)),
            pl.BlockSpec((1, 3 * D), lambda b, i: (0, 0)),
            pl.BlockSpec((D, D), lambda b, i: (0, 0)),
            pl.BlockSpec((1, D), lambda b, i: (0, 0)),
        ],
        out_specs=[node, node, node],
        out_shape=[
            jax.ShapeDtypeStruct((B, N, D), jnp.float32),
            jax.ShapeDtypeStruct((B, N, D), jnp.float32),
            jax.ShapeDtypeStruct((B, N, D), jnp.float32),
        ],
    )(Px, Ph, h, dinvb, Wx, bx, Wh, bh, Wf, bf)


def kernel(x, edge_index, Wxr, bxr, Wxz, bxz, Wxn, bxn,
           Whr, bhr, Whz, bhz, Whn, bhn, Wfc, bfc):
    row = edge_index[0]
    col = edge_index[1]
    rowb = jnp.concatenate([row, row + N])
    Wxcat = jnp.concatenate([Wxr, Wxz, Wxn], axis=1)
    Whcat = jnp.concatenate([Whr, Whz, Whn], axis=1)
    bxcat = jnp.concatenate([bxr, bxz, bxn]).reshape(1, 3 * D)
    bhcat = jnp.concatenate([bhr, bhz, bhn]).reshape(1, 3 * D)
    bfc2 = bfc.reshape(1, D)

    degs = _deg(col)
    dinv2d = _dinv(degs.reshape(NC, NPAD // D, D))
    dinvb = jnp.broadcast_to(dinv2d.reshape(NPAD)[:N, None], (N, D))

    rb = 2000
    xT = x.transpose(1, 0, 2, 3)
    xs_all = _scale(xT, dinvb, rb)

    h = jnp.zeros((B, N, D), jnp.float32)
    hs = None
    zeros2 = jnp.zeros((B, N, D), jnp.float32)
    outs = []
    for t in range(T):
        Px = _prop(xs_all[t].reshape(B * N, D), rowb, col).reshape(B, N, D)
        Ph = (_prop(hs.reshape(B * N, D), rowb, col).reshape(B, N, D)
              if t > 0 else zeros2)
        h, hs, ot = _gate(Px, Ph, h, dinvb, Wxcat, bxcat, Whcat, bhcat,
                          Wfc, bfc2, rb)
        outs.append(ot)
    return jnp.stack(outs, axis=1)


# speedup vs baseline: 26.4935x; 1.7646x over previous
import functools

import jax
import jax.numpy as jnp
from jax import lax
from jax.experimental import pallas as pl
from jax.experimental.pallas import tpu as pltpu
from jax.experimental.pallas import tpu_sc as plsc

B, T, N, E = 2, 4, 10000, 320000
D = 128
NS = 16
NC = 2
EK = 80
NPAD = 10240
ZCH = NPAD // NS

_mesh = plsc.VectorSubcoreMesh(core_axis_name="c", subcore_axis_name="s")


def _deg_body(col_hbm, deg_hbm, colv, onesv, zbuf, deg_sh):
    c = lax.axis_index("c")
    s = lax.axis_index("s")
    for j in range(ZCH // 16):
        zbuf[pl.ds(j * 16, 16)] = jnp.zeros((16,), jnp.float32)
    for j in range(EK // 16):
        onesv[pl.ds(j * 16, 16)] = jnp.ones((16,), jnp.float32)
    pltpu.sync_copy(zbuf, deg_sh.at[pl.ds(s * ZCH, ZCH)])
    plsc.subcore_barrier()
    w = s * NC + c
    ept = E // (NS * NC)
    base = w * ept

    def step(g, carry):
        pltpu.sync_copy(col_hbm.at[pl.ds(base + g * EK, EK)], colv)
        pltpu.sync_copy(onesv, deg_sh.at[colv], add=True)
        return carry

    lax.fori_loop(0, ept // EK, step, 0)
    plsc.subcore_barrier()
    pltpu.sync_copy(deg_sh.at[pl.ds(s * ZCH, ZCH)],
                    deg_hbm.at[pl.ds(c * NPAD + s * ZCH, ZCH)])


_deg = pl.kernel(
    _deg_body,
    out_type=jax.ShapeDtypeStruct((NC * NPAD,), jnp.float32),
    mesh=_mesh,
    scratch_types=[
        pltpu.VMEM((EK,), jnp.int32),
        pltpu.VMEM((EK,), jnp.float32),
        pltpu.VMEM((ZCH,), jnp.float32),
        pltpu.VMEM_SHARED((NPAD,), jnp.float32),
    ],
)


CK = 88
CHT = 228
EPAD = NS * CHT * CK
NBUF = 4
NTRASH = 10016


def _prop_body(xs_hbm, rowb_hbm, col_hbm, out_hbm,
               r0, r1, r2, r3, c0, c1, c2, c3, g0, g1, g2, g3,
               si0, si1, si2, si3, sg0, sg1, sg2, sg3,
               ss0, ss1, ss2, ss3, out_sh):
    c = lax.axis_index("c")
    s = lax.axis_index("s")
    rowv = (r0, r1, r2, r3)
    colv = (c0, c1, c2, c3)
    gbuf = (g0, g1, g2, g3)
    si = (si0, si1, si2, si3)
    sg = (sg0, sg1, sg2, sg3)
    ss = (ss0, ss1, ss2, ss3)
    rpt = N // 5

    @pl.when(s < 5)
    def _():
        pltpu.sync_copy(xs_hbm.at[pl.ds(c * N + s * rpt, rpt)],
                        out_sh.at[pl.ds(s * rpt, rpt)])

    plsc.subcore_barrier()
    ept = CHT * CK
    base = s * ept

    def block(outer, carry):
        idp = []
        for b in range(NBUF):
            off = base + (outer * NBUF + b) * CK
            i1 = pltpu.async_copy(rowb_hbm.at[pl.ds(c * EPAD + off, CK)],
                                  rowv[b], si[b])
            i2 = pltpu.async_copy(col_hbm.at[pl.ds(off, CK)], colv[b], si[b])
            idp.append((i1, i2))
        gd = []
        for b in range(NBUF):
            idp[b][0].wait()
            idp[b][1].wait()
            gd.append(pltpu.async_copy(xs_hbm.at[rowv[b]], gbuf[b], sg[b]))
        sd = []
        for b in range(NBUF):
            gd[b].wait()
            sd.append(pltpu.async_copy(gbuf[b], out_sh.at[colv[b]],
                                       ss[b], add=True))
        for b in range(NBUF):
            sd[b].wait()
        return carry

    lax.fori_loop(0, CHT // NBUF, block, 0)
    plsc.subcore_barrier()

    @pl.when(s < 5)
    def _():
        pltpu.sync_copy(out_sh.at[pl.ds(s * rpt, rpt)],
                        out_hbm.at[pl.ds(c * N + s * rpt, rpt)])


_prop = pl.kernel(
    _prop_body,
    out_type=jax.ShapeDtypeStruct((B * N, D), jnp.float32),
    mesh=_mesh,
    scratch_types=(
        [pltpu.VMEM((CK,), jnp.int32) for _ in range(NBUF)]
        + [pltpu.VMEM((CK,), jnp.int32) for _ in range(NBUF)]
        + [pltpu.VMEM((CK, D), jnp.float32) for _ in range(NBUF)]
        + [pltpu.SemaphoreType.DMA for _ in range(3 * NBUF)]
        + [pltpu.VMEM_SHARED((NTRASH, D), jnp.float32)]
    ),
)


def _dinv_body(deg_ref, out_ref):
    out_ref[...] = lax.rsqrt(deg_ref[0] + deg_ref[1] + 1.0)


def _dinv(degs):
    return pl.pallas_call(
        _dinv_body,
        out_shape=jax.ShapeDtypeStruct((NPAD // D, D), jnp.float32),
    )(degs)


def _scale_body(x_ref, d_ref, o_ref):
    o_ref[0, 0] = x_ref[0, 0] * d_ref[...]


def _scale(xT, dinvb, rb):
    return pl.pallas_call(
        _scale_body,
        grid=(T, B, N // rb),
        in_specs=[
            pl.BlockSpec((1, 1, rb, D), lambda t, b, i: (t, b, i, 0)),
            pl.BlockSpec((rb, D), lambda t, b, i: (i, 0)),
        ],
        out_specs=pl.BlockSpec((1, 1, rb, D), lambda t, b, i: (t, b, i, 0)),
        out_shape=jax.ShapeDtypeStruct((T, B, N, D), jnp.float32),
    )(xT, dinvb)


def _gate_body(px, ph, h, dv, wx, bx, wh, bh, wf, bf, hn_o, hs_o, out_o):
    d = dv[...]
    u = jnp.dot(px[0] * d, wx[...], preferred_element_type=jnp.float32) + bx[...]
    v = jnp.dot(ph[0] * d, wh[...], preferred_element_type=jnp.float32) + bh[...]
    r = jax.nn.sigmoid(u[:, :D] + v[:, :D])
    z = jax.nn.sigmoid(u[:, D:2 * D] + v[:, D:2 * D])
    n = jnp.tanh(u[:, 2 * D:] + r * v[:, 2 * D:])
    hn = (1.0 - z) * h[0] + z * n
    hn_o[0] = hn
    hs_o[0] = hn * d
    out_o[0] = jnp.dot(hn, wf[...], preferred_element_type=jnp.float32) + bf[...]


def _gate(Px, Ph, h, dinvb, Wx, bx, Wh, bh, Wf, bf, rb):
    node = pl.BlockSpec((1, rb, D), lambda b, i: (b, i, 0))
    return pl.pallas_call(
        _gate_body,
        grid=(B, N // rb),
        in_specs=[
            node, node, node,
            pl.BlockSpec((rb, D), lambda b, i: (i, 0)),
            pl.BlockSpec((D, 3 * D), lambda b, i: (0, 0)),
            pl.BlockSpec((1, 3 * D), lambda b, i: (0, 0)),
            pl.BlockSpec((D, 3 * D), lambda b, i: (0, 0)),
            pl.BlockSpec((1, 3 * D), lambda b, i: (0, 0)),
            pl.BlockSpec((D, D), lambda b, i: (0, 0)),
            pl.BlockSpec((1, D), lambda b, i: (0, 0)),
        ],
        out_specs=[node, node, node],
        out_shape=[
            jax.ShapeDtypeStruct((B, N, D), jnp.float32),
            jax.ShapeDtypeStruct((B, N, D), jnp.float32),
            jax.ShapeDtypeStruct((B, N, D), jnp.float32),
        ],
    )(Px, Ph, h, dinvb, Wx, bx, Wh, bh, Wf, bf)


def kernel(x, edge_index, Wxr, bxr, Wxz, bxz, Wxn, bxn,
           Whr, bhr, Whz, bhz, Whn, bhn, Wfc, bfc):
    row = edge_index[0]
    col = edge_index[1]
    row_p = jnp.concatenate([row, jnp.zeros(EPAD - E, jnp.int32)])
    col_p = jnp.concatenate([col, jnp.full(EPAD - E, N, jnp.int32)])
    rowb = jnp.concatenate([row_p, row_p + N])
    col3 = col_p
    Wxcat = jnp.concatenate([Wxr, Wxz, Wxn], axis=1)
    Whcat = jnp.concatenate([Whr, Whz, Whn], axis=1)
    bxcat = jnp.concatenate([bxr, bxz, bxn]).reshape(1, 3 * D)
    bhcat = jnp.concatenate([bhr, bhz, bhn]).reshape(1, 3 * D)
    bfc2 = bfc.reshape(1, D)

    degs = _deg(col)
    dinv2d = _dinv(degs.reshape(NC, NPAD // D, D))
    dinvb = jnp.broadcast_to(dinv2d.reshape(NPAD)[:N, None], (N, D))

    rb = 2000
    xT = x.transpose(1, 0, 2, 3)
    xs_all = _scale(xT, dinvb, rb)

    h = jnp.zeros((B, N, D), jnp.float32)
    hs = None
    zeros2 = jnp.zeros((B, N, D), jnp.float32)
    outs = []
    for t in range(T):
        Px = _prop(xs_all[t].reshape(B * N, D), rowb, col3).reshape(B, N, D)
        Ph = (_prop(hs.reshape(B * N, D), rowb, col3).reshape(B, N, D)
              if t > 0 else zeros2)
        h, hs, ot = _gate(Px, Ph, h, dinvb, Wxcat, bxcat, Whcat, bhcat,
                          Wfc, bfc2, rb)
        outs.append(ot)
    return jnp.stack(outs, axis=1)


# speedup vs baseline: 29.1869x; 1.1017x over previous
import functools

import jax
import jax.numpy as jnp
from jax import lax
from jax.experimental import pallas as pl
from jax.experimental.pallas import tpu as pltpu
from jax.experimental.pallas import tpu_sc as plsc

B, T, N, E = 2, 4, 10000, 320000
D = 128
NS = 16
NC = 2
EK = 80
NPAD = 10240
ZCH = NPAD // NS

_mesh = plsc.VectorSubcoreMesh(core_axis_name="c", subcore_axis_name="s")


def _deg_body(col_hbm, deg_hbm, colv, onesv, zbuf, deg_sh):
    c = lax.axis_index("c")
    s = lax.axis_index("s")
    for j in range(ZCH // 16):
        zbuf[pl.ds(j * 16, 16)] = jnp.zeros((16,), jnp.float32)
    for j in range(EK // 16):
        onesv[pl.ds(j * 16, 16)] = jnp.ones((16,), jnp.float32)
    pltpu.sync_copy(zbuf, deg_sh.at[pl.ds(s * ZCH, ZCH)])
    plsc.subcore_barrier()
    w = s * NC + c
    ept = E // (NS * NC)
    base = w * ept

    def step(g, carry):
        pltpu.sync_copy(col_hbm.at[pl.ds(base + g * EK, EK)], colv)
        pltpu.sync_copy(onesv, deg_sh.at[colv], add=True)
        return carry

    lax.fori_loop(0, ept // EK, step, 0)
    plsc.subcore_barrier()
    pltpu.sync_copy(deg_sh.at[pl.ds(s * ZCH, ZCH)],
                    deg_hbm.at[pl.ds(c * NPAD + s * ZCH, ZCH)])


_deg = pl.kernel(
    _deg_body,
    out_type=jax.ShapeDtypeStruct((NC * NPAD,), jnp.float32),
    mesh=_mesh,
    scratch_types=[
        pltpu.VMEM((EK,), jnp.int32),
        pltpu.VMEM((EK,), jnp.float32),
        pltpu.VMEM((ZCH,), jnp.float32),
        pltpu.VMEM_SHARED((NPAD,), jnp.float32),
    ],
)


CK = 88
CHT = 228
EPAD = NS * CHT * CK
NBUF = 4
NTRASH = 10016


def _prop_body(xs_hbm, rowb_hbm, col_hbm, out_hbm,
               r0, r1, r2, r3, c0, c1, c2, c3, g0, g1, g2, g3,
               si0, si1, si2, si3, sg0, sg1, sg2, sg3,
               ss0, ss1, ss2, ss3, out_sh):
    c = lax.axis_index("c")
    s = lax.axis_index("s")
    rowv = (r0, r1, r2, r3)
    colv = (c0, c1, c2, c3)
    gbuf = (g0, g1, g2, g3)
    si = (si0, si1, si2, si3)
    sg = (sg0, sg1, sg2, sg3)
    ss = (ss0, ss1, ss2, ss3)
    rpt = N // 10

    @pl.when(s < 10)
    def _():
        pltpu.sync_copy(xs_hbm.at[pl.ds(c * N + s * rpt, rpt)],
                        out_sh.at[pl.ds(s * rpt, rpt)])

    plsc.subcore_barrier()
    ept = CHT * CK
    base = s * ept

    def block(outer, carry):
        idp = []
        for b in range(NBUF):
            @pl.when(outer > 0)
            def _():
                pltpu.make_async_copy(gbuf[b], out_sh.at[colv[b]],
                                      ss[b]).wait()

            off = base + (outer * NBUF + b) * CK
            i1 = pltpu.async_copy(rowb_hbm.at[pl.ds(c * EPAD + off, CK)],
                                  rowv[b], si[b])
            i2 = pltpu.async_copy(col_hbm.at[pl.ds(off, CK)], colv[b], si[b])
            idp.append((i1, i2))
        gd = []
        for b in range(NBUF):
            idp[b][0].wait()
            idp[b][1].wait()
            gd.append(pltpu.async_copy(xs_hbm.at[rowv[b]], gbuf[b], sg[b]))
        for b in range(NBUF):
            gd[b].wait()
            pltpu.async_copy(gbuf[b], out_sh.at[colv[b]], ss[b], add=True)
        return carry

    lax.fori_loop(0, CHT // NBUF, block, 0)
    for b in range(NBUF):
        pltpu.make_async_copy(gbuf[b], out_sh.at[colv[b]], ss[b]).wait()
    plsc.subcore_barrier()

    @pl.when(s < 10)
    def _():
        pltpu.sync_copy(out_sh.at[pl.ds(s * rpt, rpt)],
                        out_hbm.at[pl.ds(c * N + s * rpt, rpt)])


_prop = pl.kernel(
    _prop_body,
    out_type=jax.ShapeDtypeStruct((B * N, D), jnp.float32),
    mesh=_mesh,
    scratch_types=(
        [pltpu.VMEM((CK,), jnp.int32) for _ in range(NBUF)]
        + [pltpu.VMEM((CK,), jnp.int32) for _ in range(NBUF)]
        + [pltpu.VMEM((CK, D), jnp.float32) for _ in range(NBUF)]
        + [pltpu.SemaphoreType.DMA for _ in range(3 * NBUF)]
        + [pltpu.VMEM_SHARED((NTRASH, D), jnp.float32)]
    ),
)


def _dinv_body(deg_ref, out_ref):
    out_ref[...] = lax.rsqrt(deg_ref[0] + deg_ref[1] + 1.0)


def _dinv(degs):
    return pl.pallas_call(
        _dinv_body,
        out_shape=jax.ShapeDtypeStruct((NPAD // D, D), jnp.float32),
    )(degs)


def _scale_body(x_ref, d_ref, o_ref):
    o_ref[0, 0] = x_ref[0, 0] * d_ref[...]


def _scale(xT, dinvb, rb):
    return pl.pallas_call(
        _scale_body,
        grid=(T, B, N // rb),
        in_specs=[
            pl.BlockSpec((1, 1, rb, D), lambda t, b, i: (t, b, i, 0)),
            pl.BlockSpec((rb, D), lambda t, b, i: (i, 0)),
        ],
        out_specs=pl.BlockSpec((1, 1, rb, D), lambda t, b, i: (t, b, i, 0)),
        out_shape=jax.ShapeDtypeStruct((T, B, N, D), jnp.float32),
    )(xT, dinvb)


def _gate_body(px, ph, h, dv, wx, bx, wh, bh, wf, bf, hn_o, hs_o, out_o):
    d = dv[...]
    u = jnp.dot(px[0] * d, wx[...], preferred_element_type=jnp.float32) + bx[...]
    v = jnp.dot(ph[0] * d, wh[...], preferred_element_type=jnp.float32) + bh[...]
    r = jax.nn.sigmoid(u[:, :D] + v[:, :D])
    z = jax.nn.sigmoid(u[:, D:2 * D] + v[:, D:2 * D])
    n = jnp.tanh(u[:, 2 * D:] + r * v[:, 2 * D:])
    hn = (1.0 - z) * h[0] + z * n
    hn_o[0] = hn
    hs_o[0] = hn * d
    out_o[0] = jnp.dot(hn, wf[...], preferred_element_type=jnp.float32) + bf[...]


def _gate(Px, Ph, h, dinvb, Wx, bx, Wh, bh, Wf, bf, rb):
    node = pl.BlockSpec((1, rb, D), lambda b, i: (b, i, 0))
    return pl.pallas_call(
        _gate_body,
        grid=(B, N // rb),
        in_specs=[
            node, node, node,
            pl.BlockSpec((rb, D), lambda b, i: (i, 0)),
            pl.BlockSpec((D, 3 * D), lambda b, i: (0, 0)),
            pl.BlockSpec((1, 3 * D), lambda b, i: (0, 0)),
            pl.BlockSpec((D, 3 * D), lambda b, i: (0, 0)),
            pl.BlockSpec((1, 3 * D), lambda b, i: (0, 0)),
            pl.BlockSpec((D, D), lambda b, i: (0, 0)),
            pl.BlockSpec((1, D), lambda b, i: (0, 0)),
        ],
        out_specs=[node, node, node],
        out_shape=[
            jax.ShapeDtypeStruct((B, N, D), jnp.float32),
            jax.ShapeDtypeStruct((B, N, D), jnp.float32),
            jax.ShapeDtypeStruct((B, N, D), jnp.float32),
        ],
    )(Px, Ph, h, dinvb, Wx, bx, Wh, bh, Wf, bf)


def kernel(x, edge_index, Wxr, bxr, Wxz, bxz, Wxn, bxn,
           Whr, bhr, Whz, bhz, Whn, bhn, Wfc, bfc):
    row = edge_index[0]
    col = edge_index[1]
    row_p = jnp.concatenate([row, jnp.zeros(EPAD - E, jnp.int32)])
    col_p = jnp.concatenate([col, jnp.full(EPAD - E, N, jnp.int32)])
    rowb = jnp.concatenate([row_p, row_p + N])
    col3 = col_p
    Wxcat = jnp.concatenate([Wxr, Wxz, Wxn], axis=1)
    Whcat = jnp.concatenate([Whr, Whz, Whn], axis=1)
    bxcat = jnp.concatenate([bxr, bxz, bxn]).reshape(1, 3 * D)
    bhcat = jnp.concatenate([bhr, bhz, bhn]).reshape(1, 3 * D)
    bfc2 = bfc.reshape(1, D)

    degs = _deg(col)
    dinv2d = _dinv(degs.reshape(NC, NPAD // D, D))
    dinvb = jnp.broadcast_to(dinv2d.reshape(NPAD)[:N, None], (N, D))

    rb = 2000
    xT = x.transpose(1, 0, 2, 3)
    xs_all = _scale(xT, dinvb, rb)

    h = jnp.zeros((B, N, D), jnp.float32)
    hs = None
    zeros2 = jnp.zeros((B, N, D), jnp.float32)
    outs = []
    for t in range(T):
        Px = _prop(xs_all[t].reshape(B * N, D), rowb, col3).reshape(B, N, D)
        Ph = (_prop(hs.reshape(B * N, D), rowb, col3).reshape(B, N, D)
              if t > 0 else zeros2)
        h, hs, ot = _gate(Px, Ph, h, dinvb, Wxcat, bxcat, Whcat, bhcat,
                          Wfc, bfc2, rb)
        outs.append(ot)
    return jnp.stack(outs, axis=1)
